# baseline (speedup 1.0000x reference)
import functools

import jax
import jax.numpy as jnp
from jax import lax
from jax.experimental import pallas as pl
from jax.experimental.pallas import tpu as pltpu
from jax.experimental.pallas import tpu_sc as plsc


def _project_body(emb_ref, w_ref, b_ref, out_ref):
    pt = lax.dot_general(
        w_ref[...],
        emb_ref[...],
        dimension_numbers=(((0,), (1,)), ((), ())),
        preferred_element_type=jnp.float32,
    )
    out_ref[...] = pt + b_ref[...]


def _project_t(emb_table, W, b):
    V = emb_table.shape[0]
    Dout = W.shape[1]
    return pl.pallas_call(
        _project_body,
        out_shape=jax.ShapeDtypeStruct((Dout, V), jnp.float32),
    )(emb_table, W, b.reshape(Dout, 1))


@functools.lru_cache(maxsize=None)
def _make_gather(V, D, Bm, Lx):
    info = plsc.get_sparse_core_info()
    NC, NS, L = info.num_cores, info.num_subcores, info.num_lanes
    NW = NC * NS
    assert Bm % NW == 0 and L == 16
    b_per_w = Bm // NW
    n_bb = b_per_w // L
    D_UNROLL = 2
    assert D % D_UNROLL == 0 and Lx % 2 == 0
    mesh = plsc.VectorSubcoreMesh(core_axis_name="c", subcore_axis_name="s")

    @functools.partial(
        pl.kernel,
        mesh=mesh,
        out_type=jax.ShapeDtypeStruct((Lx, D, Bm), jnp.float32),
        scratch_types=[
            pltpu.VMEM((D // 2 * V,), jnp.int32),
            pltpu.VMEM((b_per_w,), jnp.int32),
            pltpu.VMEM((b_per_w,), jnp.int32),
            pltpu.VMEM((D, b_per_w), jnp.float32),
            pltpu.VMEM((D, b_per_w), jnp.float32),
            pltpu.SemaphoreType.DMA,
            pltpu.SemaphoreType.DMA,
            pltpu.SemaphoreType.DMA,
            pltpu.SemaphoreType.DMA,
        ],
        compiler_params=pltpu.CompilerParams(needs_layout_passes=False),
    )
    def gather(pt_hbm, xt_hbm, out_hbm, pt_v, xv0, xv1, ob0, ob1, si0, si1, so0, so1):
        wid = lax.axis_index("s") * NC + lax.axis_index("c")
        b0 = wid * b_per_w
        xvs = (xv0, xv1)
        obs = (ob0, ob1)
        sis = (si0, si1)
        sos = (so0, so1)

        pltpu.sync_copy(pt_hbm, pt_v)
        pltpu.async_copy(xt_hbm.at[0, pl.ds(b0, b_per_w)], xv0, si0)
        pltpu.async_copy(xt_hbm.at[1, pl.ds(b0, b_per_w)], xv1, si1)

        def pair(i, carry):
            for p in range(2):
                l = 2 * i + p
                xv, ob = xvs[p], obs[p]

                pltpu.make_async_copy(
                    xt_hbm.at[0, pl.ds(0, b_per_w)], xv, sis[p]
                ).wait()

                @pl.when(l >= 2)
                def _():
                    pltpu.make_async_copy(
                        ob, out_hbm.at[0, :, pl.ds(0, b_per_w)], sos[p]
                    ).wait()

                def bb_body(bb):
                    idx = xv[pl.ds(bb * L, L)]

                    def d_body(d2, idxd):
                        w = plsc.load_gather(pt_v, [idxd])
                        lo, hi = plsc.unpack(
                            plsc.bitcast(w, jnp.bfloat16),
                            format=plsc.PackFormat.INTERLEAVED,
                            preferred_element_type=jnp.float32,
                        )
                        ob[2 * d2, pl.ds(bb * L, L)] = lo
                        ob[2 * d2 + 1, pl.ds(bb * L, L)] = hi
                        return idxd + V

                    plsc.parallel_loop(0, D // 2, unroll=D_UNROLL, carry=idx)(
                        d_body
                    )

                plsc.parallel_loop(0, n_bb)(bb_body)
                pltpu.async_copy(
                    ob, out_hbm.at[l, :, pl.ds(b0, b_per_w)], sos[p]
                )

                @pl.when(l + 2 < Lx)
                def _():
                    pltpu.async_copy(
                        xt_hbm.at[l + 2, pl.ds(b0, b_per_w)], xv, sis[p]
                    )
            return carry

        lax.fori_loop(0, Lx // 2, pair, 0)
        pltpu.make_async_copy(ob0, out_hbm.at[0, :, pl.ds(0, b_per_w)], so0).wait()
        pltpu.make_async_copy(ob1, out_hbm.at[0, :, pl.ds(0, b_per_w)], so1).wait()

    return gather


def kernel(x, emb_table, W, b):
    Bm, Lx = x.shape
    V = emb_table.shape[0]
    Dout = W.shape[1]
    ptf = _project_t(emb_table, W, b)
    pairs = ptf.astype(jnp.bfloat16).reshape(Dout // 2, 2, V).transpose(0, 2, 1)
    pt = lax.bitcast_convert_type(pairs, jnp.int32).reshape(-1)
    xt = jnp.transpose(x).astype(jnp.int32)
    out_t = _make_gather(V, Dout, Bm, Lx)(pt, xt)
    return jnp.transpose(out_t, (2, 0, 1))


# speedup vs baseline: 1.3782x; 1.3782x over previous
import functools

import jax
import jax.numpy as jnp
from jax import lax
from jax.experimental import pallas as pl
from jax.experimental.pallas import tpu as pltpu
from jax.experimental.pallas import tpu_sc as plsc


def _project_body(emb_ref, w_ref, b_ref, out_ref):
    pt = lax.dot_general(
        w_ref[...],
        emb_ref[...],
        dimension_numbers=(((0,), (1,)), ((), ())),
        preferred_element_type=jnp.float32,
    )
    out_ref[...] = pt + b_ref[...]


def _project_t(emb_table, W, b):
    V = emb_table.shape[0]
    Dout = W.shape[1]
    return pl.pallas_call(
        _project_body,
        out_shape=jax.ShapeDtypeStruct((Dout, V), jnp.float32),
    )(emb_table, W, b.reshape(Dout, 1))


@functools.lru_cache(maxsize=None)
def _make_gather(V, D, Bm, Lx):
    info = plsc.get_sparse_core_info()
    NC, NS, L = info.num_cores, info.num_subcores, info.num_lanes
    NW = NC * NS
    assert Bm % NW == 0 and L == 16
    b_per_w = Bm // NW
    n_bb = b_per_w // L
    D_UNROLL = 4
    assert D % D_UNROLL == 0 and Lx % 2 == 0
    mesh = plsc.VectorSubcoreMesh(core_axis_name="c", subcore_axis_name="s")

    @functools.partial(
        pl.kernel,
        mesh=mesh,
        out_type=jax.ShapeDtypeStruct((Lx, D, Bm), jnp.float32),
        scratch_types=[
            pltpu.VMEM((D // 2 * V,), jnp.int32),
            pltpu.VMEM((b_per_w,), jnp.int32),
            pltpu.VMEM((b_per_w,), jnp.int32),
            pltpu.VMEM((D, b_per_w), jnp.float32),
            pltpu.VMEM((D, b_per_w), jnp.float32),
            pltpu.SemaphoreType.DMA,
            pltpu.SemaphoreType.DMA,
            pltpu.SemaphoreType.DMA,
            pltpu.SemaphoreType.DMA,
        ],
        compiler_params=pltpu.CompilerParams(needs_layout_passes=False),
    )
    def gather(pt_hbm, xt_hbm, out_hbm, pt_v, xv0, xv1, ob0, ob1, si0, si1, so0, so1):
        wid = lax.axis_index("s") * NC + lax.axis_index("c")
        b0 = wid * b_per_w
        xvs = (xv0, xv1)
        obs = (ob0, ob1)
        sis = (si0, si1)
        sos = (so0, so1)

        pltpu.sync_copy(pt_hbm, pt_v)
        pltpu.async_copy(xt_hbm.at[0, pl.ds(b0, b_per_w)], xv0, si0)
        pltpu.async_copy(xt_hbm.at[1, pl.ds(b0, b_per_w)], xv1, si1)

        def pair(i, carry):
            for p in range(2):
                l = 2 * i + p
                xv, ob = xvs[p], obs[p]

                pltpu.make_async_copy(
                    xt_hbm.at[0, pl.ds(0, b_per_w)], xv, sis[p]
                ).wait()

                @pl.when(l >= 2)
                def _():
                    pltpu.make_async_copy(
                        ob, out_hbm.at[0, :, pl.ds(0, b_per_w)], sos[p]
                    ).wait()

                def bb_body(bb):
                    idx = xv[pl.ds(bb * L, L)]

                    def d_body(d2, idxd):
                        w = plsc.load_gather(pt_v, [idxd])
                        lo, hi = plsc.unpack(
                            plsc.bitcast(w, jnp.bfloat16),
                            format=plsc.PackFormat.INTERLEAVED,
                            preferred_element_type=jnp.float32,
                        )
                        ob[2 * d2, pl.ds(bb * L, L)] = lo
                        ob[2 * d2 + 1, pl.ds(bb * L, L)] = hi
                        return idxd + V

                    plsc.parallel_loop(0, D // 2, unroll=D_UNROLL, carry=idx)(
                        d_body
                    )

                plsc.parallel_loop(0, n_bb, unroll=2)(bb_body)
                pltpu.async_copy(
                    ob, out_hbm.at[l, :, pl.ds(b0, b_per_w)], sos[p]
                )

                @pl.when(l + 2 < Lx)
                def _():
                    pltpu.async_copy(
                        xt_hbm.at[l + 2, pl.ds(b0, b_per_w)], xv, sis[p]
                    )
            return carry

        lax.fori_loop(0, Lx // 2, pair, 0)
        pltpu.make_async_copy(ob0, out_hbm.at[0, :, pl.ds(0, b_per_w)], so0).wait()
        pltpu.make_async_copy(ob1, out_hbm.at[0, :, pl.ds(0, b_per_w)], so1).wait()

    return gather


def kernel(x, emb_table, W, b):
    Bm, Lx = x.shape
    V = emb_table.shape[0]
    Dout = W.shape[1]
    ptf = _project_t(emb_table, W, b)
    pairs = ptf.astype(jnp.bfloat16).reshape(Dout // 2, 2, V).transpose(0, 2, 1)
    pt = lax.bitcast_convert_type(pairs, jnp.int32).reshape(-1)
    xt = jnp.transpose(x).astype(jnp.int32)
    out_t = _make_gather(V, Dout, Bm, Lx)(pt, xt)
    return jnp.transpose(out_t, (2, 0, 1))


# speedup vs baseline: 1.4142x; 1.0261x over previous
import functools

import jax
import jax.numpy as jnp
from jax import lax
from jax.experimental import pallas as pl
from jax.experimental.pallas import tpu as pltpu
from jax.experimental.pallas import tpu_sc as plsc


def _project_body(emb_ref, w_ref, b_ref, out_ref):
    pt = lax.dot_general(
        w_ref[...],
        emb_ref[...],
        dimension_numbers=(((0,), (1,)), ((), ())),
        preferred_element_type=jnp.float32,
    )
    out_ref[...] = pt + b_ref[...]


def _project_t(emb_table, W, b):
    V = emb_table.shape[0]
    Dout = W.shape[1]
    return pl.pallas_call(
        _project_body,
        out_shape=jax.ShapeDtypeStruct((Dout, V), jnp.float32),
    )(emb_table, W, b.reshape(Dout, 1))


@functools.lru_cache(maxsize=None)
def _make_gather(V, D, Bm, Lx):
    info = plsc.get_sparse_core_info()
    NC, NS, L = info.num_cores, info.num_subcores, info.num_lanes
    NW = NC * NS
    assert Bm % NW == 0 and L == 16
    b_per_w = Bm // NW
    n_bb = b_per_w // L
    D_UNROLL = 4
    assert D % D_UNROLL == 0 and Lx % 2 == 0
    mesh = plsc.VectorSubcoreMesh(core_axis_name="c", subcore_axis_name="s")

    @functools.partial(
        pl.kernel,
        mesh=mesh,
        out_type=jax.ShapeDtypeStruct((Lx, D, Bm), jnp.float32),
        scratch_types=[
            pltpu.VMEM((D // 2 * V,), jnp.int32),
            pltpu.VMEM((b_per_w,), jnp.int32),
            pltpu.VMEM((b_per_w,), jnp.int32),
            pltpu.VMEM((D, b_per_w), jnp.float32),
            pltpu.VMEM((D, b_per_w), jnp.float32),
            pltpu.SemaphoreType.DMA,
            pltpu.SemaphoreType.DMA,
            pltpu.SemaphoreType.DMA,
            pltpu.SemaphoreType.DMA,
        ],
        compiler_params=pltpu.CompilerParams(needs_layout_passes=False),
    )
    def gather(pt_hbm, xt_hbm, out_hbm, pt_v, xv0, xv1, ob0, ob1, si0, si1, so0, so1):
        wid = lax.axis_index("s") * NC + lax.axis_index("c")
        b0 = wid * b_per_w
        xvs = (xv0, xv1)
        obs = (ob0, ob1)
        sis = (si0, si1)
        sos = (so0, so1)

        pltpu.sync_copy(pt_hbm, pt_v)
        pltpu.async_copy(xt_hbm.at[0, pl.ds(b0, b_per_w)], xv0, si0)
        pltpu.async_copy(xt_hbm.at[1, pl.ds(b0, b_per_w)], xv1, si1)

        def pair(i, carry):
            for p in range(2):
                l = 2 * i + p
                xv, ob = xvs[p], obs[p]

                pltpu.make_async_copy(
                    xt_hbm.at[0, pl.ds(0, b_per_w)], xv, sis[p]
                ).wait()

                @pl.when(l >= 2)
                def _():
                    pltpu.make_async_copy(
                        ob, out_hbm.at[0, :, pl.ds(0, b_per_w)], sos[p]
                    ).wait()

                def bb_body(bb):
                    idx = xv[pl.ds(bb * L, L)]

                    def d_body(d2, idxd):
                        w = plsc.load_gather(pt_v, [idxd])
                        lo, hi = plsc.unpack(
                            plsc.bitcast(w, jnp.bfloat16),
                            format=plsc.PackFormat.INTERLEAVED,
                            preferred_element_type=jnp.float32,
                        )
                        ob[2 * d2, pl.ds(bb * L, L)] = lo
                        ob[2 * d2 + 1, pl.ds(bb * L, L)] = hi
                        return idxd + V

                    plsc.parallel_loop(0, D // 2, unroll=D_UNROLL, carry=idx)(
                        d_body
                    )

                plsc.parallel_loop(0, n_bb, unroll=4)(bb_body)
                pltpu.async_copy(
                    ob, out_hbm.at[l, :, pl.ds(b0, b_per_w)], sos[p]
                )

                @pl.when(l + 2 < Lx)
                def _():
                    pltpu.async_copy(
                        xt_hbm.at[l + 2, pl.ds(b0, b_per_w)], xv, sis[p]
                    )
            return carry

        lax.fori_loop(0, Lx // 2, pair, 0)
        pltpu.make_async_copy(ob0, out_hbm.at[0, :, pl.ds(0, b_per_w)], so0).wait()
        pltpu.make_async_copy(ob1, out_hbm.at[0, :, pl.ds(0, b_per_w)], so1).wait()

    return gather


def kernel(x, emb_table, W, b):
    Bm, Lx = x.shape
    V = emb_table.shape[0]
    Dout = W.shape[1]
    ptf = _project_t(emb_table, W, b)
    pairs = ptf.astype(jnp.bfloat16).reshape(Dout // 2, 2, V).transpose(0, 2, 1)
    pt = lax.bitcast_convert_type(pairs, jnp.int32).reshape(-1)
    xt = jnp.transpose(x).astype(jnp.int32)
    out_t = _make_gather(V, Dout, Bm, Lx)(pt, xt)
    return jnp.transpose(out_t, (2, 0, 1))


# speedup vs baseline: 1.4280x; 1.0097x over previous
import functools

import jax
import jax.numpy as jnp
from jax import lax
from jax.experimental import pallas as pl
from jax.experimental.pallas import tpu as pltpu
from jax.experimental.pallas import tpu_sc as plsc


def _project_body(emb_ref, w_ref, b_ref, out_ref):
    pt = lax.dot_general(
        w_ref[...],
        emb_ref[...],
        dimension_numbers=(((0,), (1,)), ((), ())),
        preferred_element_type=jnp.float32,
    )
    out_ref[...] = pt + b_ref[...]


def _project_t(emb_table, W, b):
    V = emb_table.shape[0]
    Dout = W.shape[1]
    return pl.pallas_call(
        _project_body,
        out_shape=jax.ShapeDtypeStruct((Dout, V), jnp.float32),
    )(emb_table, W, b.reshape(Dout, 1))


@functools.lru_cache(maxsize=None)
def _make_gather(V, D, Bm, Lx):
    info = plsc.get_sparse_core_info()
    NC, NS, L = info.num_cores, info.num_subcores, info.num_lanes
    NW = NC * NS
    assert Bm % NW == 0 and L == 16
    b_per_w = Bm // NW
    n_bb = b_per_w // L
    D_UNROLL = 4
    assert D % D_UNROLL == 0 and Lx % 2 == 0
    mesh = plsc.VectorSubcoreMesh(core_axis_name="c", subcore_axis_name="s")

    @functools.partial(
        pl.kernel,
        mesh=mesh,
        out_type=jax.ShapeDtypeStruct((Lx, D, Bm), jnp.float32),
        scratch_types=[
            pltpu.VMEM((D // 2 * V,), jnp.int32),
            pltpu.VMEM((b_per_w,), jnp.int32),
            pltpu.VMEM((b_per_w,), jnp.int32),
            pltpu.VMEM((D, b_per_w), jnp.float32),
            pltpu.VMEM((D, b_per_w), jnp.float32),
            pltpu.SemaphoreType.DMA,
            pltpu.SemaphoreType.DMA,
            pltpu.SemaphoreType.DMA,
            pltpu.SemaphoreType.DMA,
        ],
        compiler_params=pltpu.CompilerParams(needs_layout_passes=False),
    )
    def gather(pt_hbm, xt_hbm, out_hbm, pt_v, xv0, xv1, ob0, ob1, si0, si1, so0, so1):
        wid = lax.axis_index("s") * NC + lax.axis_index("c")
        b0 = wid * b_per_w
        xvs = (xv0, xv1)
        obs = (ob0, ob1)
        sis = (si0, si1)
        sos = (so0, so1)

        pltpu.sync_copy(pt_hbm, pt_v)
        pltpu.async_copy(xt_hbm.at[0, pl.ds(b0, b_per_w)], xv0, si0)
        pltpu.async_copy(xt_hbm.at[1, pl.ds(b0, b_per_w)], xv1, si1)

        def pair(i, carry):
            for p in range(2):
                l = 2 * i + p
                xv, ob = xvs[p], obs[p]

                pltpu.make_async_copy(
                    xt_hbm.at[0, pl.ds(0, b_per_w)], xv, sis[p]
                ).wait()

                @pl.when(l >= 2)
                def _():
                    pltpu.make_async_copy(
                        ob, out_hbm.at[0, :, pl.ds(0, b_per_w)], sos[p]
                    ).wait()

                def bb_body(bb):
                    idx = xv[pl.ds(bb * L, L)]

                    def d_body(d2, idxd):
                        w = plsc.load_gather(pt_v, [idxd])
                        lo, hi = plsc.unpack(
                            plsc.bitcast(w, jnp.bfloat16),
                            format=plsc.PackFormat.INTERLEAVED,
                            preferred_element_type=jnp.float32,
                        )
                        ob[2 * d2, pl.ds(bb * L, L)] = lo
                        ob[2 * d2 + 1, pl.ds(bb * L, L)] = hi
                        return idxd + V

                    plsc.parallel_loop(0, D // 2, unroll=D_UNROLL, carry=idx)(
                        d_body
                    )

                plsc.parallel_loop(0, n_bb, unroll=8)(bb_body)
                pltpu.async_copy(
                    ob, out_hbm.at[l, :, pl.ds(b0, b_per_w)], sos[p]
                )

                @pl.when(l + 2 < Lx)
                def _():
                    pltpu.async_copy(
                        xt_hbm.at[l + 2, pl.ds(b0, b_per_w)], xv, sis[p]
                    )
            return carry

        lax.fori_loop(0, Lx // 2, pair, 0)
        pltpu.make_async_copy(ob0, out_hbm.at[0, :, pl.ds(0, b_per_w)], so0).wait()
        pltpu.make_async_copy(ob1, out_hbm.at[0, :, pl.ds(0, b_per_w)], so1).wait()

    return gather


def kernel(x, emb_table, W, b):
    Bm, Lx = x.shape
    V = emb_table.shape[0]
    Dout = W.shape[1]
    ptf = _project_t(emb_table, W, b)
    pairs = ptf.astype(jnp.bfloat16).reshape(Dout // 2, 2, V).transpose(0, 2, 1)
    pt = lax.bitcast_convert_type(pairs, jnp.int32).reshape(-1)
    xt = jnp.transpose(x).astype(jnp.int32)
    out_t = _make_gather(V, Dout, Bm, Lx)(pt, xt)
    return jnp.transpose(out_t, (2, 0, 1))


# speedup vs baseline: 1.4316x; 1.0026x over previous
import functools

import jax
import jax.numpy as jnp
from jax import lax
from jax.experimental import pallas as pl
from jax.experimental.pallas import tpu as pltpu
from jax.experimental.pallas import tpu_sc as plsc


def _project_body(emb_ref, w_ref, b_ref, out_ref):
    pt = lax.dot_general(
        w_ref[...],
        emb_ref[...],
        dimension_numbers=(((0,), (1,)), ((), ())),
        preferred_element_type=jnp.float32,
    )
    out_ref[...] = pt + b_ref[...]


def _project_t(emb_table, W, b):
    V = emb_table.shape[0]
    Dout = W.shape[1]
    return pl.pallas_call(
        _project_body,
        out_shape=jax.ShapeDtypeStruct((Dout, V), jnp.float32),
    )(emb_table, W, b.reshape(Dout, 1))


@functools.lru_cache(maxsize=None)
def _make_gather(V, D, Bm, Lx):
    info = plsc.get_sparse_core_info()
    NC, NS, L = info.num_cores, info.num_subcores, info.num_lanes
    NW = NC * NS
    assert Bm % NW == 0 and L == 16
    b_per_w = Bm // NW
    n_bb = b_per_w // L
    D_UNROLL = 4
    assert D % D_UNROLL == 0 and Lx % 2 == 0
    mesh = plsc.VectorSubcoreMesh(core_axis_name="c", subcore_axis_name="s")

    @functools.partial(
        pl.kernel,
        mesh=mesh,
        out_type=jax.ShapeDtypeStruct((Lx, D, Bm), jnp.float32),
        scratch_types=[
            pltpu.VMEM((D // 2 * V,), jnp.int32),
            pltpu.VMEM((b_per_w,), jnp.int32),
            pltpu.VMEM((b_per_w,), jnp.int32),
            pltpu.VMEM((D, b_per_w), jnp.float32),
            pltpu.VMEM((D, b_per_w), jnp.float32),
            pltpu.SemaphoreType.DMA,
            pltpu.SemaphoreType.DMA,
            pltpu.SemaphoreType.DMA,
            pltpu.SemaphoreType.DMA,
        ],
        compiler_params=pltpu.CompilerParams(needs_layout_passes=False),
    )
    def gather(pt_hbm, xt_hbm, out_hbm, pt_v, xv0, xv1, ob0, ob1, si0, si1, so0, so1):
        wid = lax.axis_index("s") * NC + lax.axis_index("c")
        b0 = wid * b_per_w
        xvs = (xv0, xv1)
        obs = (ob0, ob1)
        sis = (si0, si1)
        sos = (so0, so1)

        pltpu.sync_copy(pt_hbm, pt_v)
        pltpu.async_copy(xt_hbm.at[0, pl.ds(b0, b_per_w)], xv0, si0)
        pltpu.async_copy(xt_hbm.at[1, pl.ds(b0, b_per_w)], xv1, si1)

        def pair(i, carry):
            for p in range(2):
                l = 2 * i + p
                xv, ob = xvs[p], obs[p]

                pltpu.make_async_copy(
                    xt_hbm.at[0, pl.ds(0, b_per_w)], xv, sis[p]
                ).wait()

                @pl.when(l >= 2)
                def _():
                    pltpu.make_async_copy(
                        ob, out_hbm.at[0, :, pl.ds(0, b_per_w)], sos[p]
                    ).wait()

                def bb_body(bb):
                    idx = xv[pl.ds(bb * L, L)]

                    def d_body(d2, idxd):
                        w = plsc.load_gather(pt_v, [idxd])
                        lo, hi = plsc.unpack(
                            plsc.bitcast(w, jnp.bfloat16),
                            format=plsc.PackFormat.INTERLEAVED,
                            preferred_element_type=jnp.float32,
                        )
                        ob[2 * d2, pl.ds(bb * L, L)] = lo
                        ob[2 * d2 + 1, pl.ds(bb * L, L)] = hi
                        return idxd + V

                    plsc.parallel_loop(0, D // 2, unroll=D_UNROLL, carry=idx)(
                        d_body
                    )

                plsc.parallel_loop(0, n_bb, unroll=16)(bb_body)
                pltpu.async_copy(
                    ob, out_hbm.at[l, :, pl.ds(b0, b_per_w)], sos[p]
                )

                @pl.when(l + 2 < Lx)
                def _():
                    pltpu.async_copy(
                        xt_hbm.at[l + 2, pl.ds(b0, b_per_w)], xv, sis[p]
                    )
            return carry

        lax.fori_loop(0, Lx // 2, pair, 0)
        pltpu.make_async_copy(ob0, out_hbm.at[0, :, pl.ds(0, b_per_w)], so0).wait()
        pltpu.make_async_copy(ob1, out_hbm.at[0, :, pl.ds(0, b_per_w)], so1).wait()

    return gather


def kernel(x, emb_table, W, b):
    Bm, Lx = x.shape
    V = emb_table.shape[0]
    Dout = W.shape[1]
    ptf = _project_t(emb_table, W, b)
    pairs = ptf.astype(jnp.bfloat16).reshape(Dout // 2, 2, V).transpose(0, 2, 1)
    pt = lax.bitcast_convert_type(pairs, jnp.int32).reshape(-1)
    xt = jnp.transpose(x).astype(jnp.int32)
    out_t = _make_gather(V, Dout, Bm, Lx)(pt, xt)
    return jnp.transpose(out_t, (2, 0, 1))
